# baseline (speedup 1.0000x reference)
import functools

import jax
import jax.numpy as jnp
from jax.experimental import pallas as pl
from jax.experimental.pallas import tpu as pltpu

_HI = jax.lax.Precision.HIGHEST


def _softplus(x):
    return jnp.maximum(x, 0.0) + jnp.log(1.0 + jnp.exp(-jnp.abs(x)))


def _hidot(a, b):
    return jnp.dot(a, b, precision=_HI, preferred_element_type=jnp.float32)


def _gcnet_kernel(
        img_ref,
        mask_ref,
        l1_ref,
        l1i_ref,
        nw_ref,
        l2_ref,
        l2i_ref,
        g3_ref,
        qmat_ref,
        qt_ref,
        p_ref,
        p8t_ref,
        s8_ref,
        k8m_ref,
        selimg_ref,
        seldir_ref,
        repf_ref,
        s3_ref,
        qtile_ref,
        rep8c_ref,
        normal_ref,
        shading_ref,
        dirs1_ref,
        intens1_ref,
        dirs2_ref,
        intens2_ref):
    B, C, H, W = img_ref.shape
    F = C // 3
    inv_hw = 1.0 / (H * W)

    def img_at(b):
        return img_ref[b]

    mask = mask_ref[...].reshape(B * H, W)
    qmat, g3 = qmat_ref[...], g3_ref[...]

    def to_mats(vals):
        spread = _hidot(vals, s3_ref[...]) * qtile_ref[...]
        return _hidot(p_ref[...], spread)

    l1d, l1i = l1_ref[...], l1i_ref[...]
    l2d, l2i = l2_ref[...], l2i_ref[...]

    def blockdiag(a3):
        return _hidot(_hidot(qmat, a3), qt_ref[...]) * g3

    w1 = jnp.concatenate([blockdiag(l1d[:, 0:3]),
                          blockdiag(l1i[:, 0:3])], axis=0)
    b1 = jnp.concatenate([_hidot(qmat, l1d[:, 3:4]),
                          _hidot(qmat, l1i[:, 3:4])], axis=0)
    w2 = jnp.concatenate([blockdiag(l2d[:, 0:3]),
                          blockdiag(l2i[:, 0:3])], axis=0)
    b2m = jnp.concatenate([_hidot(qmat, l2d[:, 3:4]),
                           _hidot(qmat, l2i[:, 3:4])], axis=0)
    w2n = jnp.concatenate([_hidot(qmat, l2d[:, 4:7]),
                           _hidot(qmat, l2i[:, 4:7])], axis=0)
    w2s = jnp.concatenate([_hidot(qmat, l2d[:, 7:8]) * repf_ref[...],
                           _hidot(qmat, l2i[:, 7:8]) * repf_ref[...]],
                          axis=0)
    wn3 = _hidot(nw_ref[...], selimg_ref[...])
    wnd3 = _hidot(nw_ref[...], seldir_ref[...])
    kw = _hidot(_hidot(p8t_ref[...], wn3), s8_ref[...]) * k8m_ref[...]

    pools, mpools = [], []
    for b in range(B):
        pw = jnp.sum(img_at(b), axis=1)
        pools.append(jnp.sum(pw, axis=-1, keepdims=True) * inv_hw)
        mpools.append(jnp.mean(mask[b * H:(b + 1) * H, :], keepdims=True))
    pool = jnp.concatenate(pools, axis=1)
    mpool = jnp.concatenate(mpools, axis=1)

    h1 = (jnp.dot(w1, pool, preferred_element_type=jnp.float32)
          + b1 * mpool)
    d1, i1 = h1[0:C], h1[C:2 * C]
    nrm1 = jnp.sqrt(_hidot(g3, d1 * d1))
    dirs1 = d1 / (nrm1 + 1e-8)
    intens1 = _softplus(i1) + 0.2

    inv = pl.reciprocal(intens1 + 1e-8, approx=True)
    dir3 = jnp.dot(wnd3, dirs1,
                   preferred_element_type=jnp.float32)
    dir24 = _hidot(p8t_ref[...], dir3)

    m1 = to_mats(jnp.concatenate([dirs1, intens1], axis=1))
    for b in range(B):
        dirs1_ref[b] = m1[:, 3 * b:3 * b + 3]
        intens1_ref[b] = m1[:, 3 * (B + b):3 * (B + b) + 3]

    np_cols, sp_cols = [], []
    for b in range(B):
        imgb = img_at(b)
        inv192 = _hidot(rep8c_ref[...], inv[:, b:b + 1])
        dcol = dir24[:, b:b + 1]
        dmat = m1[:, 3 * b:3 * b + 3]
        dm0 = dmat[:, 0:1].reshape(F, 1, 1)
        dm1 = dmat[:, 1:2].reshape(F, 1, 1)
        dm2 = dmat[:, 2:3].reshape(F, 1, 1)
        np_acc = jnp.zeros((3, 8, W), jnp.float32)
        sp_acc = jnp.zeros((F, 8, W), jnp.float32)
        for k in range(H // 8):
            sl = slice(k * 8, (k + 1) * 8)
            chunk = imgb[:, sl, :].reshape(C * 8, W) * inv192
            raw = (jnp.dot(kw, chunk,
                           preferred_element_type=jnp.float32)
                   + dcol)
            t3 = jnp.tanh(raw).reshape(3, 8, W)
            ssum = jnp.sum(t3 * t3, axis=0)
            scale = (jax.lax.rsqrt(ssum + 1e-8)
                     * mask[b * H + k * 8:b * H + (k + 1) * 8, :])
            normal_k = t3 * scale
            normal_ref[b, :, sl, :] = normal_k
            np_acc = np_acc + normal_k

            shad_k = (dm0 * normal_k[0:1] + dm1 * normal_k[1:2]
                      + dm2 * normal_k[2:3])
            shad_k = jnp.clip(shad_k, 0.0, 1.0)
            shading_ref[b, :, sl, :] = shad_k
            sp_acc = sp_acc + shad_k
        np_cols.append(
            jnp.sum(np_acc, axis=(1, 2), keepdims=True).reshape(3, 1) * inv_hw)
        sp_cols.append(
            jnp.sum(sp_acc, axis=(1, 2), keepdims=True).reshape(F, 1) * inv_hw)

    npool = jnp.concatenate(np_cols, axis=1)
    sp = jnp.concatenate(sp_cols, axis=1)

    h2 = (jnp.dot(w2, pool, preferred_element_type=jnp.float32)
          + b2m * mpool
          + jnp.dot(w2n, npool, preferred_element_type=jnp.float32)
          + _hidot(w2s, sp))
    d2, i2 = h2[0:C], h2[C:2 * C]
    nrm2 = jnp.sqrt(_hidot(g3, d2 * d2))
    dirs2 = d2 / (nrm2 + 1e-8)
    intens2 = _softplus(i2) + 0.2
    m2 = to_mats(jnp.concatenate([dirs2, intens2], axis=1))
    for b in range(B):
        dirs2_ref[b] = m2[:, 3 * b:3 * b + 3]
        intens2_ref[b] = m2[:, 3 * (B + b):3 * (B + b) + 3]


def kernel(img, mask, l1_wd, l1_wi, n_w, l2_wd, l2_wi):
    N, c3f, H, W = img.shape
    F = c3f // 3
    C = 3 * F
    f32 = jnp.float32
    img = img.astype(f32)
    mask = mask.astype(f32)

    import numpy as np
    nf32 = np.float32
    eyeF = np.eye(F, dtype=nf32)
    eye3 = np.eye(3, dtype=nf32)
    g3 = jnp.asarray(np.kron(eyeF, np.ones((3, 3), nf32)))
    qmat_n = np.kron(np.ones((F, 1), nf32), eye3)
    pmat_n = np.kron(eyeF, np.ones((1, 3), nf32))
    qmat, qt = jnp.asarray(qmat_n), jnp.asarray(qmat_n.T)
    pmat = jnp.asarray(pmat_n)
    p8t = jnp.asarray(np.kron(eye3, np.ones((8, 1), nf32)))
    s8 = jnp.asarray(np.kron(np.eye(C, dtype=nf32),
                             np.ones((1, 8), nf32)))
    k8m = jnp.asarray(np.kron(np.ones((3, C), nf32),
                              np.eye(8, dtype=nf32)))
    selimg_n = np.zeros((6 * F, C), nf32)
    seldir_n = np.zeros((6 * F, C), nf32)
    for f in range(F):
        for j in range(3):
            selimg_n[6 * f + j, 3 * f + j] = 1.0
            seldir_n[6 * f + 3 + j, 3 * f + j] = 1.0
    selimg, seldir = jnp.asarray(selimg_n), jnp.asarray(seldir_n)
    repf = jnp.asarray(np.kron(eyeF, np.ones((3, 1), nf32)))
    B = 8
    s3 = jnp.asarray(np.kron(np.eye(2 * B, dtype=nf32),
                             np.ones((1, 3), nf32)))
    qtile = jnp.asarray(np.tile(qmat_n, (1, 2 * B)))
    rep8c = jnp.asarray(np.kron(np.eye(C, dtype=nf32),
                                np.ones((8, 1), nf32)))

    def cspec(shape):
        return pl.BlockSpec(shape, lambda n: (0,) * len(shape))

    Np = N
    if N % B:
        Np = (N + B - 1) // B * B
        img = jnp.concatenate(
            [img, jnp.zeros((Np - N, c3f, H, W), f32)], axis=0)
        mask = jnp.concatenate(
            [mask, jnp.zeros((Np - N, 1, H, W), f32)], axis=0)

    outs = pl.pallas_call(
        _gcnet_kernel,
        grid=(Np // B,),
        in_specs=[
            pl.BlockSpec((B, C, H, W), lambda n: (n, 0, 0, 0)),
            pl.BlockSpec((B, 1, H, W), lambda n: (n, 0, 0, 0)),
            cspec((3, 4)), cspec((3, 4)),
            cspec((3, 6 * F)),
            cspec((3, 8)), cspec((3, 8)),
            cspec((C, C)),
            cspec((C, 3)), cspec((3, C)),
            cspec((F, C)),
            cspec((C, 3)),
            cspec((C, 8 * C)), cspec((C, 8 * C)),
            cspec((6 * F, C)), cspec((6 * F, C)),
            cspec((C, F)),
            cspec((2 * B, 6 * B)), cspec((C, 6 * B)),
            cspec((8 * C, C)),
        ],
        out_specs=[
            pl.BlockSpec((B, 3, H, W), lambda n: (n, 0, 0, 0)),
            pl.BlockSpec((B, F, H, W), lambda n: (n, 0, 0, 0)),
            pl.BlockSpec((B, F, 3), lambda n: (n, 0, 0)),
            pl.BlockSpec((B, F, 3), lambda n: (n, 0, 0)),
            pl.BlockSpec((B, F, 3), lambda n: (n, 0, 0)),
            pl.BlockSpec((B, F, 3), lambda n: (n, 0, 0)),
        ],
        out_shape=[
            jax.ShapeDtypeStruct((Np, 3, H, W), f32),
            jax.ShapeDtypeStruct((Np, F, H, W), f32),
            jax.ShapeDtypeStruct((Np, F, 3), f32),
            jax.ShapeDtypeStruct((Np, F, 3), f32),
            jax.ShapeDtypeStruct((Np, F, 3), f32),
            jax.ShapeDtypeStruct((Np, F, 3), f32),
        ],
        compiler_params=pltpu.CompilerParams(
            dimension_semantics=("parallel",)),
    )(img, mask, l1_wd.astype(f32), l1_wi.astype(f32),
      n_w.astype(f32), l2_wd.astype(f32), l2_wi.astype(f32),
      g3, qmat, qt, pmat, p8t, s8, k8m, selimg, seldir, repf, s3, qtile,
      rep8c)

    normal, shading, dirs1, intens1, dirs2, intens2 = (
        o[:N] if Np != N else o for o in outs)
    return {
        'prev_dirs': dirs1,
        'prev_intens': intens1,
        'prev_normal': normal,
        'prev_shading': shading,
        'dirs': dirs2,
        'intens': intens2,
    }


# speedup vs baseline: 1.2779x; 1.2779x over previous
import functools

import jax
import jax.numpy as jnp
from jax.experimental import pallas as pl
from jax.experimental.pallas import tpu as pltpu

_HI = jax.lax.Precision.HIGHEST


def _softplus(x):
    return jnp.maximum(x, 0.0) + jnp.log(1.0 + jnp.exp(-jnp.abs(x)))


def _hidot(a, b):
    return jnp.dot(a, b, precision=_HI, preferred_element_type=jnp.float32)


def _gcnet_kernel(
        img_ref,
        mask_ref,
        l1_ref,
        l1i_ref,
        nw_ref,
        l2_ref,
        l2i_ref,
        g3_ref,
        qmat_ref,
        qt_ref,
        p_ref,
        p8t_ref,
        s8_ref,
        k8m_ref,
        selimg_ref,
        seldir_ref,
        repf_ref,
        s3_ref,
        qtile_ref,
        normal_ref,
        shading_ref,
        dirs1_ref,
        intens1_ref,
        dirs2_ref,
        intens2_ref):
    B, C, H, W = img_ref.shape
    F = C // 3
    inv_hw = 1.0 / (H * W)

    def img_at(b):
        return img_ref[b]

    mask = mask_ref[...].reshape(B * H, W)
    qmat, g3 = qmat_ref[...], g3_ref[...]

    def to_mats(vals):
        spread = _hidot(vals, s3_ref[...]) * qtile_ref[...]
        return _hidot(p_ref[...], spread)

    l1d, l1i = l1_ref[...], l1i_ref[...]
    l2d, l2i = l2_ref[...], l2i_ref[...]

    def blockdiag(a3):
        return _hidot(_hidot(qmat, a3), qt_ref[...]) * g3

    w1 = jnp.concatenate([blockdiag(l1d[:, 0:3]),
                          blockdiag(l1i[:, 0:3])], axis=0)
    b1 = jnp.concatenate([_hidot(qmat, l1d[:, 3:4]),
                          _hidot(qmat, l1i[:, 3:4])], axis=0)
    w2 = jnp.concatenate([blockdiag(l2d[:, 0:3]),
                          blockdiag(l2i[:, 0:3])], axis=0)
    b2m = jnp.concatenate([_hidot(qmat, l2d[:, 3:4]),
                           _hidot(qmat, l2i[:, 3:4])], axis=0)
    w2n = jnp.concatenate([_hidot(qmat, l2d[:, 4:7]),
                           _hidot(qmat, l2i[:, 4:7])], axis=0)
    w2s = jnp.concatenate([_hidot(qmat, l2d[:, 7:8]) * repf_ref[...],
                           _hidot(qmat, l2i[:, 7:8]) * repf_ref[...]],
                          axis=0)
    wn3 = _hidot(nw_ref[...], selimg_ref[...])
    wnd3 = _hidot(nw_ref[...], seldir_ref[...])
    kw = _hidot(_hidot(p8t_ref[...], wn3), s8_ref[...]) * k8m_ref[...]

    pools, mpools = [], []
    for b in range(B):
        pw = jnp.sum(img_at(b), axis=1)
        pools.append(jnp.sum(pw, axis=-1, keepdims=True) * inv_hw)
        mpools.append(jnp.mean(mask[b * H:(b + 1) * H, :], keepdims=True))
    pool = jnp.concatenate(pools, axis=1)
    mpool = jnp.concatenate(mpools, axis=1)

    h1 = (jnp.dot(w1, pool, preferred_element_type=jnp.float32)
          + b1 * mpool)
    d1, i1 = h1[0:C], h1[C:2 * C]
    nrm1 = jnp.sqrt(_hidot(g3, d1 * d1))
    dirs1 = d1 / (nrm1 + 1e-8)
    intens1 = _softplus(i1) + 0.2

    inv = pl.reciprocal(intens1 + 1e-8, approx=True)
    dir3 = jnp.dot(wnd3, dirs1,
                   preferred_element_type=jnp.float32)
    dir24 = _hidot(p8t_ref[...], dir3)

    m1 = to_mats(jnp.concatenate([dirs1, intens1], axis=1))
    for b in range(B):
        dirs1_ref[b] = m1[:, 3 * b:3 * b + 3]
        intens1_ref[b] = m1[:, 3 * (B + b):3 * (B + b) + 3]

    np_cols, sp_cols = [], []
    for b in range(B):
        imgn = img_at(b) * inv[:, b:b + 1].reshape(C, 1, 1)
        dcol = dir24[:, b:b + 1]
        dmat = m1[:, 3 * b:3 * b + 3]
        dm0 = dmat[:, 0:1].reshape(F, 1, 1)
        dm1 = dmat[:, 1:2].reshape(F, 1, 1)
        dm2 = dmat[:, 2:3].reshape(F, 1, 1)
        np_acc = jnp.zeros((3, 8, W), jnp.float32)
        sp_acc = jnp.zeros((F, 8, W), jnp.float32)
        for k in range(H // 8):
            sl = slice(k * 8, (k + 1) * 8)
            chunk = imgn[:, sl, :].reshape(C * 8, W)
            raw = (jnp.dot(kw, chunk,
                           preferred_element_type=jnp.float32)
                   + dcol)
            t3 = jnp.tanh(raw).reshape(3, 8, W)
            ssum = jnp.sum(t3 * t3, axis=0)
            scale = (jax.lax.rsqrt(ssum + 1e-8)
                     * mask[b * H + k * 8:b * H + (k + 1) * 8, :])
            normal_k = t3 * scale
            normal_ref[b, :, sl, :] = normal_k
            np_acc = np_acc + normal_k

            shad_k = (dm0 * normal_k[0:1] + dm1 * normal_k[1:2]
                      + dm2 * normal_k[2:3])
            shad_k = jnp.clip(shad_k, 0.0, 1.0)
            shading_ref[b, :, sl, :] = shad_k
            sp_acc = sp_acc + shad_k
        np_cols.append(
            jnp.sum(np_acc, axis=(1, 2), keepdims=True).reshape(3, 1) * inv_hw)
        sp_cols.append(
            jnp.sum(sp_acc, axis=(1, 2), keepdims=True).reshape(F, 1) * inv_hw)

    npool = jnp.concatenate(np_cols, axis=1)
    sp = jnp.concatenate(sp_cols, axis=1)

    h2 = (jnp.dot(w2, pool, preferred_element_type=jnp.float32)
          + b2m * mpool
          + jnp.dot(w2n, npool, preferred_element_type=jnp.float32)
          + _hidot(w2s, sp))
    d2, i2 = h2[0:C], h2[C:2 * C]
    nrm2 = jnp.sqrt(_hidot(g3, d2 * d2))
    dirs2 = d2 / (nrm2 + 1e-8)
    intens2 = _softplus(i2) + 0.2
    m2 = to_mats(jnp.concatenate([dirs2, intens2], axis=1))
    for b in range(B):
        dirs2_ref[b] = m2[:, 3 * b:3 * b + 3]
        intens2_ref[b] = m2[:, 3 * (B + b):3 * (B + b) + 3]


def kernel(img, mask, l1_wd, l1_wi, n_w, l2_wd, l2_wi):
    N, c3f, H, W = img.shape
    F = c3f // 3
    C = 3 * F
    f32 = jnp.float32
    img = img.astype(f32)
    mask = mask.astype(f32)

    import numpy as np
    nf32 = np.float32
    eyeF = np.eye(F, dtype=nf32)
    eye3 = np.eye(3, dtype=nf32)
    g3 = jnp.asarray(np.kron(eyeF, np.ones((3, 3), nf32)))
    qmat_n = np.kron(np.ones((F, 1), nf32), eye3)
    pmat_n = np.kron(eyeF, np.ones((1, 3), nf32))
    qmat, qt = jnp.asarray(qmat_n), jnp.asarray(qmat_n.T)
    pmat = jnp.asarray(pmat_n)
    p8t = jnp.asarray(np.kron(eye3, np.ones((8, 1), nf32)))
    s8 = jnp.asarray(np.kron(np.eye(C, dtype=nf32),
                             np.ones((1, 8), nf32)))
    k8m = jnp.asarray(np.kron(np.ones((3, C), nf32),
                              np.eye(8, dtype=nf32)))
    selimg_n = np.zeros((6 * F, C), nf32)
    seldir_n = np.zeros((6 * F, C), nf32)
    for f in range(F):
        for j in range(3):
            selimg_n[6 * f + j, 3 * f + j] = 1.0
            seldir_n[6 * f + 3 + j, 3 * f + j] = 1.0
    selimg, seldir = jnp.asarray(selimg_n), jnp.asarray(seldir_n)
    repf = jnp.asarray(np.kron(eyeF, np.ones((3, 1), nf32)))
    B = 8
    s3 = jnp.asarray(np.kron(np.eye(2 * B, dtype=nf32),
                             np.ones((1, 3), nf32)))
    qtile = jnp.asarray(np.tile(qmat_n, (1, 2 * B)))

    def cspec(shape):
        return pl.BlockSpec(shape, lambda n: (0,) * len(shape))

    Np = N
    if N % B:
        Np = (N + B - 1) // B * B
        img = jnp.concatenate(
            [img, jnp.zeros((Np - N, c3f, H, W), f32)], axis=0)
        mask = jnp.concatenate(
            [mask, jnp.zeros((Np - N, 1, H, W), f32)], axis=0)

    outs = pl.pallas_call(
        _gcnet_kernel,
        grid=(Np // B,),
        in_specs=[
            pl.BlockSpec((B, C, H, W), lambda n: (n, 0, 0, 0)),
            pl.BlockSpec((B, 1, H, W), lambda n: (n, 0, 0, 0)),
            cspec((3, 4)), cspec((3, 4)),
            cspec((3, 6 * F)),
            cspec((3, 8)), cspec((3, 8)),
            cspec((C, C)),
            cspec((C, 3)), cspec((3, C)),
            cspec((F, C)),
            cspec((C, 3)),
            cspec((C, 8 * C)), cspec((C, 8 * C)),
            cspec((6 * F, C)), cspec((6 * F, C)),
            cspec((C, F)),
            cspec((2 * B, 6 * B)), cspec((C, 6 * B)),
        ],
        out_specs=[
            pl.BlockSpec((B, 3, H, W), lambda n: (n, 0, 0, 0)),
            pl.BlockSpec((B, F, H, W), lambda n: (n, 0, 0, 0)),
            pl.BlockSpec((B, F, 3), lambda n: (n, 0, 0)),
            pl.BlockSpec((B, F, 3), lambda n: (n, 0, 0)),
            pl.BlockSpec((B, F, 3), lambda n: (n, 0, 0)),
            pl.BlockSpec((B, F, 3), lambda n: (n, 0, 0)),
        ],
        out_shape=[
            jax.ShapeDtypeStruct((Np, 3, H, W), f32),
            jax.ShapeDtypeStruct((Np, F, H, W), f32),
            jax.ShapeDtypeStruct((Np, F, 3), f32),
            jax.ShapeDtypeStruct((Np, F, 3), f32),
            jax.ShapeDtypeStruct((Np, F, 3), f32),
            jax.ShapeDtypeStruct((Np, F, 3), f32),
        ],
        compiler_params=pltpu.CompilerParams(
            dimension_semantics=("parallel",)),
    )(img, mask, l1_wd.astype(f32), l1_wi.astype(f32),
      n_w.astype(f32), l2_wd.astype(f32), l2_wi.astype(f32),
      g3, qmat, qt, pmat, p8t, s8, k8m, selimg, seldir, repf, s3, qtile)

    normal, shading, dirs1, intens1, dirs2, intens2 = (
        o[:N] if Np != N else o for o in outs)
    return {
        'prev_dirs': dirs1,
        'prev_intens': intens1,
        'prev_normal': normal,
        'prev_shading': shading,
        'dirs': dirs2,
        'intens': intens2,
    }
